# baseline (speedup 1.0000x reference)
import functools

import jax
import jax.numpy as jnp
from jax import lax
from jax.experimental import pallas as pl
from jax.experimental.pallas import tpu as pltpu
from jax.experimental.pallas import tpu_sc as plsc

MAX_POS = 1024
HIDDEN = 768
EPS = 1e-12

NC = 2
NS = 16
NW = NC * NS

CHUNK = 128
HW2 = HIDDEN // 2


def _sc_gather(tbl, idx_all, n_rows):
    rows_per_w = n_rows // NW
    n_chunks = rows_per_w // CHUNK
    mesh = plsc.VectorSubcoreMesh(core_axis_name="c", subcore_axis_name="s")

    @functools.partial(
        pl.kernel,
        out_type=jax.ShapeDtypeStruct((n_rows, HW2), jnp.int32),
        mesh=mesh,
        scratch_types=[
            pltpu.VMEM((CHUNK,), jnp.int32),
            pltpu.VMEM((CHUNK,), jnp.int32),
            pltpu.VMEM((CHUNK, HW2), jnp.int32),
            pltpu.VMEM((CHUNK, HW2), jnp.int32),
            pltpu.SemaphoreType.DMA,
            pltpu.SemaphoreType.DMA,
        ],
    )
    def k(tbl_hbm, idx_hbm, out_hbm, idx0, idx1, rows0, rows1, sem0, sem1):
        wid = lax.axis_index("s") * NC + lax.axis_index("c")
        base = wid * rows_per_w
        idx_v = (idx0, idx1)
        rows_v = (rows0, rows1)
        sems = (sem0, sem1)
        copies = [None, None]
        pltpu.sync_copy(idx_hbm.at[pl.ds(base, CHUNK)], idx0)
        copies[0] = pltpu.async_copy(tbl_hbm.at[idx0], rows0, sem0)
        for c in range(n_chunks):
            cur = c % 2
            nxt = (c + 1) % 2
            if c + 1 < n_chunks:
                off_n = base + (c + 1) * CHUNK
                pltpu.sync_copy(idx_hbm.at[pl.ds(off_n, CHUNK)], idx_v[nxt])
                copies[nxt] = pltpu.async_copy(
                    tbl_hbm.at[idx_v[nxt]], rows_v[nxt], sems[nxt]
                )
            copies[cur].wait()
            pltpu.sync_copy(rows_v[cur], out_hbm.at[pl.ds(base + c * CHUNK, CHUNK)])

    return k(tbl, idx_all)


BT = 512


def _tc_body(rows_ref, g_ref, bt_ref, wt_ref, b_ref, out_ref):
    r = rows_ref[...].astype(jnp.float32)
    emb = (r[0] + r[1]) + (r[2] + r[3])
    mean = jnp.mean(emb, axis=-1, keepdims=True)
    d = emb - mean
    var = jnp.mean(d * d, axis=-1, keepdims=True)
    nrm = d * lax.rsqrt(var + EPS) * g_ref[...] + bt_ref[...]
    out_ref[...] = (
        jnp.dot(
            nrm.astype(jnp.bfloat16), wt_ref[...],
            preferred_element_type=jnp.float32,
        )
        + b_ref[...]
    )


def _tc_ln_mlp(rows, gamma, beta, w_t, b):
    n_tok = rows.shape[1]
    return pl.pallas_call(
        _tc_body,
        grid=(n_tok // BT,),
        in_specs=[
            pl.BlockSpec((4, BT, HIDDEN), lambda i: (0, i, 0)),
            pl.BlockSpec((1, HIDDEN), lambda i: (0, 0)),
            pl.BlockSpec((1, HIDDEN), lambda i: (0, 0)),
            pl.BlockSpec((HIDDEN, HIDDEN), lambda i: (0, 0)),
            pl.BlockSpec((1, HIDDEN), lambda i: (0, 0)),
        ],
        out_specs=pl.BlockSpec((BT, HIDDEN), lambda i: (i, 0)),
        out_shape=jax.ShapeDtypeStruct((n_tok, HIDDEN), jnp.float32),
    )(rows, gamma, beta, w_t, b)


def kernel(bbox, x_table, y_table, ln_gamma, ln_beta, W, b):
    batch, seq, _ = bbox.shape
    n_tok = batch * seq
    idx = bbox.reshape(n_tok, 4).astype(jnp.int32)
    idx_all = jnp.concatenate(
        [idx[:, 0], idx[:, 1] + MAX_POS, idx[:, 2], idx[:, 3] + MAX_POS], axis=0
    )
    tbl = jnp.concatenate([x_table, y_table], axis=0).astype(jnp.bfloat16)
    tbl_w = lax.bitcast_convert_type(tbl.reshape(-1, HW2, 2), jnp.int32)
    rows_w = _sc_gather(tbl_w, idx_all, 4 * n_tok)
    rows = lax.bitcast_convert_type(rows_w, jnp.bfloat16)
    rows = rows.reshape(4, n_tok, HIDDEN)
    out = _tc_ln_mlp(
        rows,
        ln_gamma.reshape(1, HIDDEN),
        ln_beta.reshape(1, HIDDEN),
        W.T.astype(jnp.bfloat16),
        b.reshape(1, HIDDEN),
    )
    return out.reshape(batch, seq, HIDDEN)


# speedup vs baseline: 5.0613x; 5.0613x over previous
import functools

import jax
import jax.numpy as jnp
from jax import lax
from jax.experimental import pallas as pl
from jax.experimental.pallas import tpu as pltpu
from jax.experimental.pallas import tpu_sc as plsc

MAX_POS = 1024
HIDDEN = 768
EPS = 1e-12

NC = 2
NS = 16
NW = NC * NS

CHUNK = 128
HW2 = HIDDEN // 2


def _sc_gather(tbl, idx_all, n_rows):
    rows_per_w = n_rows // NW
    n_chunks = rows_per_w // CHUNK
    mesh = plsc.VectorSubcoreMesh(core_axis_name="c", subcore_axis_name="s")

    @functools.partial(
        pl.kernel,
        out_type=jax.ShapeDtypeStruct((n_rows, HW2), jnp.float32),
        mesh=mesh,
        scratch_types=[
            pltpu.VMEM((CHUNK,), jnp.int32),
            pltpu.VMEM((CHUNK,), jnp.int32),
            pltpu.VMEM((CHUNK, HW2), jnp.float32),
            pltpu.VMEM((CHUNK, HW2), jnp.float32),
            pltpu.SemaphoreType.DMA,
            pltpu.SemaphoreType.DMA,
        ],
    )
    def k(tbl_hbm, idx_hbm, out_hbm, idx0, idx1, rows0, rows1, sem0, sem1):
        wid = lax.axis_index("s") * NC + lax.axis_index("c")
        base = wid * rows_per_w
        idx_v = (idx0, idx1)
        rows_v = (rows0, rows1)
        sems = (sem0, sem1)
        copies = [None, None]
        pltpu.sync_copy(idx_hbm.at[pl.ds(base, CHUNK)], idx0)
        copies[0] = pltpu.async_copy(tbl_hbm.at[idx0], rows0, sem0)
        for c in range(n_chunks):
            cur = c % 2
            nxt = (c + 1) % 2
            if c + 1 < n_chunks:
                off_n = base + (c + 1) * CHUNK
                pltpu.sync_copy(idx_hbm.at[pl.ds(off_n, CHUNK)], idx_v[nxt])
                copies[nxt] = pltpu.async_copy(
                    tbl_hbm.at[idx_v[nxt]], rows_v[nxt], sems[nxt]
                )
            copies[cur].wait()
            pltpu.sync_copy(rows_v[cur], out_hbm.at[pl.ds(base + c * CHUNK, CHUNK)])

    return k(tbl, idx_all)


BT = 512


def _tc_body(rows_ref, g_ref, bt_ref, wt_ref, b_ref, out_ref):
    w = lax.bitcast_convert_type(rows_ref[...], jnp.uint32)
    lo = lax.bitcast_convert_type(w << 16, jnp.float32)
    hi = lax.bitcast_convert_type(w & jnp.uint32(0xFFFF0000), jnp.float32)
    lo = (lo[0] + lo[1]) + (lo[2] + lo[3])
    hi = (hi[0] + hi[1]) + (hi[2] + hi[3])
    emb = jnp.concatenate([lo, hi], axis=-1)
    mean = jnp.mean(emb, axis=-1, keepdims=True)
    d = emb - mean
    var = jnp.mean(d * d, axis=-1, keepdims=True)
    nrm = d * lax.rsqrt(var + EPS) * g_ref[...] + bt_ref[...]
    out_ref[...] = (
        jnp.dot(
            nrm.astype(jnp.bfloat16), wt_ref[...],
            preferred_element_type=jnp.float32,
        )
        + b_ref[...]
    )


def _tc_ln_mlp(rows, gamma, beta, w_t, b):
    n_tok = rows.shape[1]
    return pl.pallas_call(
        _tc_body,
        grid=(n_tok // BT,),
        in_specs=[
            pl.BlockSpec((4, BT, HW2), lambda i: (0, i, 0)),
            pl.BlockSpec((1, HIDDEN), lambda i: (0, 0)),
            pl.BlockSpec((1, HIDDEN), lambda i: (0, 0)),
            pl.BlockSpec((HIDDEN, HIDDEN), lambda i: (0, 0)),
            pl.BlockSpec((1, HIDDEN), lambda i: (0, 0)),
        ],
        out_specs=pl.BlockSpec((BT, HIDDEN), lambda i: (i, 0)),
        out_shape=jax.ShapeDtypeStruct((n_tok, HIDDEN), jnp.float32),
    )(rows, gamma, beta, w_t, b)


def kernel(bbox, x_table, y_table, ln_gamma, ln_beta, W, b):
    batch, seq, _ = bbox.shape
    n_tok = batch * seq
    idx = bbox.reshape(n_tok, 4).astype(jnp.int32)
    idx_all = jnp.concatenate(
        [idx[:, 0], idx[:, 1] + MAX_POS, idx[:, 2], idx[:, 3] + MAX_POS], axis=0
    )
    tbl = jnp.concatenate([x_table, y_table], axis=0).astype(jnp.bfloat16)
    bits = lax.bitcast_convert_type(tbl, jnp.uint16).astype(jnp.uint32)
    words = bits[:, :HW2] | (bits[:, HW2:] << 16)
    tbl_w = lax.bitcast_convert_type(words, jnp.float32)
    rows = _sc_gather(tbl_w, idx_all, 4 * n_tok)
    rows = rows.reshape(4, n_tok, HW2)
    out = _tc_ln_mlp(
        rows,
        ln_gamma.reshape(1, HIDDEN),
        ln_beta.reshape(1, HIDDEN),
        W.T.astype(jnp.bfloat16),
        b.reshape(1, HIDDEN),
    )
    return out.reshape(batch, seq, HIDDEN)
